# Initial kernel scaffold; baseline (speedup 1.0000x reference)
import jax
import jax.numpy as jnp
from jax.experimental import pallas as pl


def kernel(ts_emb_seq, codebook):
    raise NotImplementedError("write your pallas kernel here")


# speedup vs baseline: 1.0743x; 1.0743x over previous
import functools

import jax
import jax.numpy as jnp
from jax.experimental import pallas as pl
from jax.experimental.pallas import tpu as pltpu

K_NODES = 8192
D_LATENT = 32
ALPHA = 5.0
TILE = 128


def _som_body(z_ref, cb_ref, q1_ref, q2_ref, bmu_ref, zq_ref):
    z = z_ref[...]
    cb = cb_ref[...]

    zsq = jnp.sum(z * z, axis=1, keepdims=True)
    csq = jax.lax.dot_general(
        jnp.ones((1, D_LATENT), jnp.float32), cb * cb,
        (((1,), (1,)), ((), ())),
        preferred_element_type=jnp.float32,
        precision=jax.lax.Precision.HIGHEST)
    zc = jax.lax.dot_general(
        z, cb, (((1,), (1,)), ((), ())),
        preferred_element_type=jnp.float32,
        precision=jax.lax.Precision.DEFAULT)
    d = jnp.maximum(zsq + csq - 2.0 * zc, 0.0)

    m = jnp.min(d, axis=1, keepdims=True)
    lane = jax.lax.broadcasted_iota(jnp.int32, d.shape, 1)
    bmu = jnp.min(jnp.where(d == m, lane, K_NODES), axis=1, keepdims=True)
    bmu_ref[...] = bmu

    onehot = (lane == bmu).astype(jnp.float32)
    zq_ref[...] = jax.lax.dot_general(
        onehot, cb, (((1,), (0,)), ((), ())),
        preferred_element_type=jnp.float32,
        precision=jax.lax.Precision.HIGHEST)

    t = 1.0 + d / ALPHA
    r = 1.0 / t
    qu = r * r * r
    s = jnp.sum(qu, axis=1, keepdims=True)
    q = qu / s
    q1_ref[...] = q
    q2_ref[...] = q


@jax.jit
def kernel(ts_emb_seq, codebook):
    b, t_max, d_latent = ts_emb_seq.shape
    n = b * t_max
    z = ts_emb_seq.reshape(n, d_latent)

    grid = (n // TILE,)
    q1, q2, bmu, zq = pl.pallas_call(
        _som_body,
        grid=grid,
        in_specs=[
            pl.BlockSpec((TILE, D_LATENT), lambda i: (i, 0)),
            pl.BlockSpec((K_NODES, D_LATENT), lambda i: (0, 0)),
        ],
        out_specs=[
            pl.BlockSpec((TILE, K_NODES), lambda i: (i, 0)),
            pl.BlockSpec((TILE, K_NODES), lambda i: (i, 0)),
            pl.BlockSpec((TILE, 1), lambda i: (i, 0)),
            pl.BlockSpec((TILE, D_LATENT), lambda i: (i, 0)),
        ],
        out_shape=[
            jax.ShapeDtypeStruct((n, K_NODES), jnp.float32),
            jax.ShapeDtypeStruct((n, K_NODES), jnp.float32),
            jax.ShapeDtypeStruct((n, 1), jnp.int32),
            jax.ShapeDtypeStruct((n, D_LATENT), jnp.float32),
        ],
    )(z, codebook)

    return (q1, q2, bmu.reshape(n), zq)


# speedup vs baseline: 1.9909x; 1.8531x over previous
import functools

import jax
import jax.numpy as jnp
from jax.experimental import pallas as pl
from jax.experimental.pallas import tpu as pltpu

K_NODES = 8192
D_LATENT = 32
ALPHA = 5.0
TILE = 128


def _som_body(z_ref, cb_ref, q1_ref, q2_ref, bmu_ref, zq_ref, csq_ref):
    i = pl.program_id(0)
    cb = cb_ref[...]

    @pl.when(i == 0)
    def _():
        csq_ref[...] = jax.lax.dot_general(
            jnp.ones((1, D_LATENT), jnp.float32), cb * cb,
            (((1,), (1,)), ((), ())),
            preferred_element_type=jnp.float32,
            precision=jax.lax.Precision.HIGHEST)

    z = z_ref[...]
    zsq = jnp.sum(z * z, axis=1, keepdims=True)
    csq = csq_ref[...]
    zc = jax.lax.dot_general(
        z, cb, (((1,), (1,)), ((), ())),
        preferred_element_type=jnp.float32,
        precision=jax.lax.Precision.DEFAULT)
    d = jnp.maximum((zsq + csq) - 2.0 * zc, 0.0)

    m = jnp.min(d, axis=1, keepdims=True)
    lane = jax.lax.broadcasted_iota(jnp.int32, d.shape, 1)
    bmu = jnp.min(jnp.where(d == m, lane, K_NODES), axis=1, keepdims=True)
    bmu_ref[...] = bmu

    onehot = (lane == bmu).astype(jnp.float32)
    cb_hi = cb.astype(jnp.bfloat16).astype(jnp.float32)
    cb_lo = cb - cb_hi
    zq_hi = jax.lax.dot_general(
        onehot, cb_hi, (((1,), (0,)), ((), ())),
        preferred_element_type=jnp.float32,
        precision=jax.lax.Precision.DEFAULT)
    zq_lo = jax.lax.dot_general(
        onehot, cb_lo, (((1,), (0,)), ((), ())),
        preferred_element_type=jnp.float32,
        precision=jax.lax.Precision.DEFAULT)
    zq_ref[...] = zq_hi + zq_lo

    t = 1.0 + d / ALPHA
    r = 1.0 / t
    qu = r * r * r
    s = jnp.sum(qu, axis=1, keepdims=True)
    q = qu * (1.0 / s)
    q1_ref[...] = q
    q2_ref[...] = q


@jax.jit
def kernel(ts_emb_seq, codebook):
    b, t_max, d_latent = ts_emb_seq.shape
    n = b * t_max
    z = ts_emb_seq.reshape(n, d_latent)

    grid = (n // TILE,)
    q1, q2, bmu, zq = pl.pallas_call(
        _som_body,
        grid=grid,
        in_specs=[
            pl.BlockSpec((TILE, D_LATENT), lambda i: (i, 0)),
            pl.BlockSpec((K_NODES, D_LATENT), lambda i: (0, 0)),
        ],
        out_specs=[
            pl.BlockSpec((TILE, K_NODES), lambda i: (i, 0)),
            pl.BlockSpec((TILE, K_NODES), lambda i: (i, 0)),
            pl.BlockSpec((TILE, 1), lambda i: (i, 0)),
            pl.BlockSpec((TILE, D_LATENT), lambda i: (i, 0)),
        ],
        out_shape=[
            jax.ShapeDtypeStruct((n, K_NODES), jnp.float32),
            jax.ShapeDtypeStruct((n, K_NODES), jnp.float32),
            jax.ShapeDtypeStruct((n, 1), jnp.int32),
            jax.ShapeDtypeStruct((n, D_LATENT), jnp.float32),
        ],
        scratch_shapes=[pltpu.VMEM((1, K_NODES), jnp.float32)],
    )(z, codebook)

    return (q1, q2, bmu.reshape(n), zq)
